# initial kernel scaffold (unmeasured)
import jax
import jax.numpy as jnp
from jax import lax
from jax.experimental import pallas as pl
from jax.experimental.pallas import tpu as pltpu

N_DEV = 16


def kernel(x, w_mat):
    m_full, k_per = x.shape
    k_full, n = w_mat.shape
    m_per = m_full // N_DEV

    def body(x_ref, w_ref, out_ref, xrow_ref, send_sems, recv_sems):
        me = lax.axis_index("i")

        xrow_ref[:, pl.ds(me * k_per, k_per)] = x_ref[pl.ds(me * m_per, m_per), :]

        copies = []
        for d in range(1, N_DEV):
            dst = lax.rem(me + d, N_DEV)
            rdma = pltpu.make_async_remote_copy(
                src_ref=x_ref.at[pl.ds(dst * m_per, m_per), :],
                dst_ref=xrow_ref.at[:, pl.ds(me * k_per, k_per)],
                send_sem=send_sems.at[d],
                recv_sem=recv_sems.at[d],
                device_id=(dst,),
                device_id_type=pl.DeviceIdType.MESH,
            )
            rdma.start()
            copies.append(rdma)

        for rdma in copies:
            rdma.wait()

        out_ref[...] = jnp.dot(
            xrow_ref[...], w_ref[...], preferred_element_type=jnp.float32
        )

    return pl.pallas_call(
        body,
        out_shape=jax.ShapeDtypeStruct((m_per, n), jnp.float32),
        in_specs=[
            pl.BlockSpec(memory_space=pltpu.VMEM),
            pl.BlockSpec(memory_space=pltpu.VMEM),
        ],
        out_specs=pl.BlockSpec(memory_space=pltpu.VMEM),
        scratch_shapes=[
            pltpu.VMEM((m_per, k_full), jnp.float32),
            pltpu.SemaphoreType.DMA((N_DEV,)),
            pltpu.SemaphoreType.DMA((N_DEV,)),
        ],
        compiler_params=pltpu.CompilerParams(collective_id=0),
    )(x, w_mat)


# baseline (device time: 28880 ns/iter reference)
import jax
import jax.numpy as jnp
from jax import lax
from jax.experimental import pallas as pl
from jax.experimental.pallas import tpu as pltpu

N_DEV = 16


def kernel(x, w_mat):
    m_full, k_per = x.shape
    k_full, n = w_mat.shape
    m_per = m_full // N_DEV

    def body(x_ref, w_ref, out_ref, xrow_ref, send_sems, recv_sems):
        me = lax.axis_index("i")

        xrow_ref[:, pl.ds(me * k_per, k_per)] = x_ref[pl.ds(me * m_per, m_per), :]

        copies = []
        for d in range(1, N_DEV):
            dst = lax.rem(me + d, N_DEV)
            rdma = pltpu.make_async_remote_copy(
                src_ref=x_ref.at[pl.ds(dst * m_per, m_per), :],
                dst_ref=xrow_ref.at[:, pl.ds(me * k_per, k_per)],
                send_sem=send_sems.at[d],
                recv_sem=recv_sems.at[d],
                device_id=(dst,),
                device_id_type=pl.DeviceIdType.MESH,
            )
            rdma.start()
            copies.append(rdma)

        for rdma in copies:
            rdma.wait()

        out_ref[...] = jnp.dot(
            xrow_ref[...], w_ref[...], preferred_element_type=jnp.float32
        )

    return pl.pallas_call(
        body,
        out_shape=jax.ShapeDtypeStruct((m_per, n), jnp.float32),
        in_specs=[
            pl.BlockSpec(memory_space=pltpu.VMEM),
            pl.BlockSpec(memory_space=pltpu.VMEM),
        ],
        out_specs=pl.BlockSpec(memory_space=pltpu.VMEM),
        scratch_shapes=[
            pltpu.VMEM((m_per, k_full), jnp.float32),
            pltpu.SemaphoreType.DMA((N_DEV,)),
            pltpu.SemaphoreType.DMA((N_DEV,)),
        ],
    )(x, w_mat)
